# baseline (speedup 1.0000x reference)
import jax
import jax.numpy as jnp
from jax import lax
from jax.experimental import pallas as pl
from jax.experimental.pallas import tpu as pltpu
from jax.experimental.pallas import tpu_sc as plsc

NS = 16
L = 16
B = 16384
BT = B // NS
V = 5000
D = 64
G = D // L
VP = 5120
CR = VP // D
RT = VP // NS
CRT = CR // NS
SCALE = 1.0 / B
ZERO16 = (0.0,) * L


def _acc_rows(cnt_vec, tbl_v, row0, nk, acc):
    acc = list(acc)
    for kk in range(nk):
        c = lax.broadcast(cnt_vec[kk], (L,))
        h = (kk % 2) * G
        for g in range(G):
            acc[h + g] = acc[h + g] + c * tbl_v[row0 + kk, pl.ds(g * L, L)]
    return tuple(acc)


def _sc_body(idx_hbm, table_hbm, out_hbm, idx_v, cnt_v, cntm_v, tbl_v,
             acc_v, part_v, shcnt_v, shacc_v, semt, semm):
    sid = lax.axis_index("s")
    nlast = V - RT * (NS - 1)
    @pl.when(sid < NS - 1)
    def _():
        pltpu.async_copy(table_hbm.at[pl.ds(sid * RT, RT)], tbl_v, semt)

    @pl.when(sid == NS - 1)
    def _():
        pltpu.async_copy(table_hbm.at[pl.ds((NS - 1) * RT, nlast)],
                         tbl_v.at[pl.ds(0, nlast)], semt)

    pltpu.sync_copy(idx_hbm.at[sid], idx_v)

    def zbody(i, _):
        for j in range(G):
            cnt_v[i, pl.ds(j * L, L)] = jnp.zeros((L,), jnp.float32)
        return 0

    lax.fori_loop(0, CR, zbody, 0)
    ones = jnp.ones((L,), jnp.float32)

    def hbody(i, _):
        iv = idx_v[pl.ds(i * L, L)]
        r = lax.shift_right_logical(iv, 6)
        c = lax.bitwise_and(iv, D - 1)
        plsc.addupdate_scatter(cnt_v, [r, c], ones)
        return 0

    lax.fori_loop(0, BT // L, hbody, 0)
    pltpu.sync_copy(cnt_v, shcnt_v.at[sid])
    plsc.subcore_barrier()

    mcopies = [
        pltpu.async_copy(shcnt_v.at[s, pl.ds(sid * CRT, CRT)],
                         cntm_v.at[s], semm)
        for s in range(NS)
    ]
    for c in mcopies:
        c.wait()
    for r in range(CRT):
        for j in range(G):
            s = cntm_v[0, r, pl.ds(j * L, L)]
            for t in range(1, NS):
                s = s + cntm_v[t, r, pl.ds(j * L, L)]
            cnt_v[r, pl.ds(j * L, L)] = s

    acc0 = tuple(jnp.zeros((L,), jnp.float32) for _ in range(2 * G))

    @pl.when(sid < NS - 1)
    def _():
        pltpu.make_async_copy(
            table_hbm.at[pl.ds(sid * RT, RT)], tbl_v, semt).wait()

        def gbody(gi, a):
            cv = cnt_v[lax.div(gi, G), pl.ds(lax.rem(gi, G) * L, L)]
            return _acc_rows(cv, tbl_v, gi * L, L, a)

        acc = lax.fori_loop(0, RT // L, gbody, acc0)
        for k in range(G):
            acc_v[pl.ds(k * L, L)] = acc[k] + acc[G + k]

    @pl.when(sid == NS - 1)
    def _():
        nfull = nlast // L
        ntail = nlast - nfull * L
        pltpu.make_async_copy(
            table_hbm.at[pl.ds((NS - 1) * RT, nlast)],
            tbl_v.at[pl.ds(0, nlast)], semt).wait()

        def gbody(gi, a):
            cv = cnt_v[lax.div(gi, G), pl.ds(lax.rem(gi, G) * L, L)]
            return _acc_rows(cv, tbl_v, gi * L, L, a)

        acc = lax.fori_loop(0, nfull, gbody, acc0)
        cv = cnt_v[nfull // G, pl.ds((nfull % G) * L, L)]
        acc = _acc_rows(cv, tbl_v, nfull * L, ntail, acc)
        for k in range(G):
            acc_v[pl.ds(k * L, L)] = acc[k] + acc[G + k]

    pltpu.sync_copy(acc_v, shacc_v.at[sid])
    plsc.subcore_barrier()

    @pl.when(sid == 0)
    def _():
        pltpu.sync_copy(shacc_v, part_v)
        for k in range(G):
            s = part_v[0, pl.ds(k * L, L)]
            for t in range(1, NS):
                s = s + part_v[t, pl.ds(k * L, L)]
            acc_v[pl.ds(k * L, L)] = s * SCALE
        pltpu.sync_copy(acc_v, out_hbm)


def kernel(idxs, table):
    idx2 = idxs.reshape(NS, BT)
    mesh = plsc.VectorSubcoreMesh(
        core_axis_name="c", subcore_axis_name="s", num_cores=1)
    f = pl.kernel(
        _sc_body,
        out_type=jax.ShapeDtypeStruct((D,), jnp.float32),
        mesh=mesh,
        scratch_types=[
            pltpu.VMEM((BT,), jnp.int32),
            pltpu.VMEM((CR, D), jnp.float32),
            pltpu.VMEM((NS, CRT, D), jnp.float32),
            pltpu.VMEM((RT, D), jnp.float32),
            pltpu.VMEM((D,), jnp.float32),
            pltpu.VMEM((NS, D), jnp.float32),
            pltpu.VMEM_SHARED((NS, CR, D), jnp.float32),
            pltpu.VMEM_SHARED((NS, D), jnp.float32),
            pltpu.SemaphoreType.DMA,
            pltpu.SemaphoreType.DMA,
        ],
        compiler_params=pltpu.CompilerParams(
            use_tc_tiling_on_sc=False, needs_layout_passes=False),
    )
    return f(idx2, table)
